# baseline (speedup 1.0000x reference)
import functools

import jax
import jax.numpy as jnp
from jax import lax
from jax.experimental import pallas as pl
from jax.experimental.pallas import tpu as pltpu
from jax.experimental.pallas import tpu_sc as plsc

K = 8192
D = 64
BETA = 0.25

BM = 256
BK = 2048


def _argmin_body(flat_ref, w_ref, rn_ref, wn_ref, idx_ref, best_ref, bidx_ref):
    j = pl.program_id(1)
    nk = pl.num_programs(1)

    mm = lax.dot_general(flat_ref[...], w_ref[...], (((1,), (1,)), ((), ())),
                         preferred_element_type=jnp.float32)
    d2 = (rn_ref[...] - 2.0 * mm) + wn_ref[...]
    dist = jnp.sqrt(jnp.maximum(d2, 0.0))

    lm = jnp.min(dist, axis=1, keepdims=True)
    col = jax.lax.broadcasted_iota(jnp.int32, dist.shape, 1) + j * BK
    li = jnp.min(jnp.where(dist == lm, col, jnp.int32(2**30)),
                 axis=1, keepdims=True)

    @pl.when(j == 0)
    def _init():
        best_ref[...] = lm.astype(jnp.bfloat16).astype(jnp.float32)
        bidx_ref[...] = li

    @pl.when(j > 0)
    def _update():
        better = lm < best_ref[...]
        newb = jnp.where(better, lm, best_ref[...])
        bidx_ref[...] = jnp.where(better, li, bidx_ref[...])
        best_ref[...] = newb.astype(jnp.bfloat16).astype(jnp.float32)

    @pl.when(j == nk - 1)
    def _store():
        idx_ref[...] = bidx_ref[...]


def _argmin_call(flat, w):
    n = flat.shape[0]
    rn = jnp.sum(flat * flat, axis=1, keepdims=True)
    wn = jnp.sum(w * w, axis=1)[None, :]
    return pl.pallas_call(
        _argmin_body,
        grid=(n // BM, K // BK),
        in_specs=[
            pl.BlockSpec((BM, D), lambda i, j: (i, 0)),
            pl.BlockSpec((BK, D), lambda i, j: (j, 0)),
            pl.BlockSpec((BM, 1), lambda i, j: (i, 0)),
            pl.BlockSpec((1, BK), lambda i, j: (0, j)),
        ],
        out_specs=pl.BlockSpec((BM, 1), lambda i, j: (i, 0)),
        out_shape=jax.ShapeDtypeStruct((n, 1), jnp.int32),
        scratch_shapes=[
            pltpu.VMEM((BM, 1), jnp.float32),
            pltpu.VMEM((BM, 1), jnp.int32),
        ],
    )(flat.astype(jnp.bfloat16), w, rn, wn)


def _gather_call(w, idx):
    info = plsc.get_sparse_core_info()
    nw = info.num_cores * info.num_subcores
    n = idx.shape[0]
    b_per_w = n // nw
    mesh = plsc.VectorSubcoreMesh(core_axis_name="c", subcore_axis_name="s")

    chunk = 128
    nchunk = b_per_w // chunk

    @functools.partial(
        pl.kernel,
        mesh=mesh,
        out_type=jax.ShapeDtypeStruct((n, D), jnp.float32),
        scratch_types=[
            pltpu.VMEM((nchunk, chunk), jnp.int32),
            pltpu.VMEM((b_per_w, D), jnp.float32),
            pltpu.SemaphoreType.DMA,
        ],
        compiler_params=pltpu.CompilerParams(use_tc_tiling_on_sc=False),
    )
    def gather(w_hbm, idx_hbm, out_hbm, idx_v, rows_v, sem):
        wid = lax.axis_index("s") * info.num_cores + lax.axis_index("c")
        base = wid * b_per_w
        for c in range(nchunk):
            pltpu.sync_copy(idx_hbm.at[pl.ds(base + c * chunk, chunk)],
                            idx_v.at[c])
        descs = [
            pltpu.async_copy(w_hbm.at[idx_v.at[c]],
                             rows_v.at[pl.ds(c * chunk, chunk)], sem)
            for c in range(nchunk)
        ]
        for d_ in descs:
            d_.wait()
        pltpu.sync_copy(rows_v, out_hbm.at[pl.ds(base, b_per_w)])

    return gather(w, idx)


def _st_loss_body(e_ref, q_ref, st_ref, loss_ref):
    e = e_ref[...]
    q = q_ref[...]
    t = q - e
    st_ref[...] = e + t
    m = jnp.mean(t * t)
    loss_ref[0, 0] = m * BETA + m


def _st_loss_call(flat, q):
    n = flat.shape[0]
    return pl.pallas_call(
        _st_loss_body,
        out_shape=(
            jax.ShapeDtypeStruct((n, D), jnp.float32),
            jax.ShapeDtypeStruct((1, 1), jnp.float32),
        ),
        out_specs=(
            pl.BlockSpec((n, D), lambda: (0, 0)),
            pl.BlockSpec(memory_space=pltpu.SMEM),
        ),
    )(flat, q)


def kernel(encoding, W):
    shape = encoding.shape
    flat = encoding.reshape(-1, D)
    idx = _argmin_call(flat, W)
    q = _gather_call(W, idx.reshape(-1))
    st, loss = _st_loss_call(flat, q)
    return idx, st.reshape(shape), loss.reshape(())


# speedup vs baseline: 1.0196x; 1.0196x over previous
import functools

import jax
import jax.numpy as jnp
from jax import lax
from jax.experimental import pallas as pl
from jax.experimental.pallas import tpu as pltpu
from jax.experimental.pallas import tpu_sc as plsc

K = 8192
D = 64
BETA = 0.25

BM = 256
BK = 2048


def _argmin_body(flat_ref, w_ref, rn_ref, wn_ref, idx_ref, best_ref, bidx_ref):
    i = pl.program_id(0)
    j = pl.program_id(1)
    nk = pl.num_programs(1)

    fb = flat_ref[pl.ds(i * BM, BM), :].astype(jnp.bfloat16)
    w = w_ref[pl.ds(j * BK, BK), :]
    mm = lax.dot_general(fb, w, (((1,), (1,)), ((), ())),
                         preferred_element_type=jnp.float32)
    d2 = (rn_ref[...] - 2.0 * mm) + wn_ref[pl.ds(0, 1), pl.ds(j * BK, BK)]
    dist = jnp.sqrt(jnp.maximum(d2, 0.0))

    lm = jnp.min(dist, axis=1, keepdims=True)
    col = jax.lax.broadcasted_iota(jnp.int32, dist.shape, 1) + j * BK
    li = jnp.min(jnp.where(dist == lm, col, jnp.int32(2**30)),
                 axis=1, keepdims=True)

    @pl.when(j == 0)
    def _init():
        best_ref[...] = lm.astype(jnp.bfloat16).astype(jnp.float32)
        bidx_ref[...] = li

    @pl.when(j > 0)
    def _update():
        better = lm < best_ref[...]
        newb = jnp.where(better, lm, best_ref[...])
        bidx_ref[...] = jnp.where(better, li, bidx_ref[...])
        best_ref[...] = newb.astype(jnp.bfloat16).astype(jnp.float32)

    @pl.when(j == nk - 1)
    def _store():
        idx_ref[...] = bidx_ref[...]


def _argmin_call(flat, w):
    n = flat.shape[0]
    rn = jnp.sum(flat * flat, axis=1, keepdims=True)
    wn = jnp.sum(w * w, axis=1)[None, :]
    return pl.pallas_call(
        _argmin_body,
        grid=(n // BM, K // BK),
        in_specs=[
            pl.BlockSpec((n, D), lambda i, j: (0, 0)),
            pl.BlockSpec((K, D), lambda i, j: (0, 0)),
            pl.BlockSpec((BM, 1), lambda i, j: (i, 0)),
            pl.BlockSpec((1, K), lambda i, j: (0, 0)),
        ],
        out_specs=pl.BlockSpec((BM, 1), lambda i, j: (i, 0)),
        out_shape=jax.ShapeDtypeStruct((n, 1), jnp.int32),
        scratch_shapes=[
            pltpu.VMEM((BM, 1), jnp.float32),
            pltpu.VMEM((BM, 1), jnp.int32),
        ],
    )(flat, w, rn, wn)


def _gather_call(w, idx):
    info = plsc.get_sparse_core_info()
    nw = info.num_cores * info.num_subcores
    n = idx.shape[0]
    b_per_w = n // nw
    mesh = plsc.VectorSubcoreMesh(core_axis_name="c", subcore_axis_name="s")

    chunk = 128
    nchunk = b_per_w // chunk

    @functools.partial(
        pl.kernel,
        mesh=mesh,
        out_type=jax.ShapeDtypeStruct((n, D), jnp.float32),
        scratch_types=[
            pltpu.VMEM((nchunk, chunk), jnp.int32),
            pltpu.VMEM((b_per_w, D), jnp.float32),
            pltpu.SemaphoreType.DMA,
        ],
        compiler_params=pltpu.CompilerParams(use_tc_tiling_on_sc=False),
    )
    def gather(w_hbm, idx_hbm, out_hbm, idx_v, rows_v, sem):
        wid = lax.axis_index("s") * info.num_cores + lax.axis_index("c")
        base = wid * b_per_w
        for c in range(nchunk):
            pltpu.sync_copy(idx_hbm.at[pl.ds(base + c * chunk, chunk)],
                            idx_v.at[c])
        descs = [
            pltpu.async_copy(w_hbm.at[idx_v.at[c]],
                             rows_v.at[pl.ds(c * chunk, chunk)], sem)
            for c in range(nchunk)
        ]
        for d_ in descs:
            d_.wait()
        pltpu.sync_copy(rows_v, out_hbm.at[pl.ds(base, b_per_w)])

    return gather(w, idx)


def _st_loss_body(e_ref, q_ref, st_ref, loss_ref):
    e = e_ref[...]
    q = q_ref[...]
    t = q - e
    st_ref[...] = e + t
    m = jnp.mean(t * t)
    loss_ref[0, 0] = m * BETA + m


def _st_loss_call(flat, q):
    n = flat.shape[0]
    return pl.pallas_call(
        _st_loss_body,
        out_shape=(
            jax.ShapeDtypeStruct((n, D), jnp.float32),
            jax.ShapeDtypeStruct((1, 1), jnp.float32),
        ),
        out_specs=(
            pl.BlockSpec((n, D), lambda: (0, 0)),
            pl.BlockSpec(memory_space=pltpu.SMEM),
        ),
    )(flat, q)


def kernel(encoding, W):
    shape = encoding.shape
    flat = encoding.reshape(-1, D)
    idx = _argmin_call(flat, W)
    q = _gather_call(W, idx.reshape(-1))
    st, loss = _st_loss_call(flat, q)
    return idx, st.reshape(shape), loss.reshape(())
